# Initial kernel scaffold; baseline (speedup 1.0000x reference)
import jax
import jax.numpy as jnp
from jax.experimental import pallas as pl


def kernel(x, table, W, b):
    raise NotImplementedError("write your pallas kernel here")


# speedup vs baseline: 2.8290x; 2.8290x over previous
import functools

import jax
import jax.numpy as jnp
from jax import lax
from jax.experimental import pallas as pl
from jax.experimental.pallas import tpu as pltpu
from jax.experimental.pallas import tpu_sc as plsc

VOCAB_P1 = 100001
EMB = 128
BATCH = 16384
WINDOW = 5
N_CLASS = 50

NC = 2
NS = 16
NW = NC * NS

TOTAL_ROWS = BATCH * WINDOW
ROWS_PER_W = TOTAL_ROWS // NW
CHUNK = 128
NCHUNK = ROWS_PER_W // CHUNK


def _gather_body(idx_hbm, table_hbm, out_hbm, idx_v, rows_v, sem):
  wid = lax.axis_index("s") * NC + lax.axis_index("c")
  row_base = wid * ROWS_PER_W
  pltpu.sync_copy(idx_hbm, idx_v)
  for j in range(NCHUNK):
    pltpu.async_copy(table_hbm.at[idx_v.at[wid * NCHUNK + j]], rows_v, sem).wait()
    pltpu.sync_copy(rows_v, out_hbm.at[pl.ds(row_base + j * CHUNK, CHUNK)])


_sc_gather = functools.partial(
    pl.kernel,
    out_type=jax.ShapeDtypeStruct((TOTAL_ROWS, EMB), jnp.float32),
    mesh=plsc.VectorSubcoreMesh(core_axis_name="c", subcore_axis_name="s"),
    scratch_types=[
        pltpu.VMEM((TOTAL_ROWS // CHUNK, CHUNK), jnp.int32),
        pltpu.VMEM((CHUNK, EMB), jnp.float32),
        pltpu.SemaphoreType.DMA,
    ],
)(_gather_body)


BM = 1024


def _matmul_body(flat_ref, w_ref, b_ref, out_ref):
  acc = lax.dot_general(
      flat_ref[...], w_ref[...],
      dimension_numbers=(((1,), (1,)), ((), ())),
      preferred_element_type=jnp.float32,
  )
  out_ref[...] = acc + b_ref[...]


def _tc_matmul(flat, w, b2d):
  return pl.pallas_call(
      _matmul_body,
      grid=(BATCH // BM,),
      in_specs=[
          pl.BlockSpec((BM, WINDOW * EMB), lambda i: (i, 0)),
          pl.BlockSpec((N_CLASS, WINDOW * EMB), lambda i: (0, 0)),
          pl.BlockSpec((1, N_CLASS), lambda i: (0, 0)),
      ],
      out_specs=pl.BlockSpec((BM, N_CLASS), lambda i: (i, 0)),
      out_shape=jax.ShapeDtypeStruct((BATCH, N_CLASS), jnp.float32),
  )(flat, w, b2d)


@jax.jit
def kernel(x, table, W, b):
  idx = x.astype(jnp.int32).reshape(TOTAL_ROWS // CHUNK, CHUNK)
  gathered = _sc_gather(idx, table)
  flat = gathered.reshape(BATCH, WINDOW * EMB)
  return _tc_matmul(flat, W, b.reshape(1, N_CLASS))


# speedup vs baseline: 4.7421x; 1.6762x over previous
import functools

import jax
import jax.numpy as jnp
from jax import lax
from jax.experimental import pallas as pl
from jax.experimental.pallas import tpu as pltpu
from jax.experimental.pallas import tpu_sc as plsc

VOCAB_P1 = 100001
EMB = 128
BATCH = 16384
WINDOW = 5
N_CLASS = 50

NC = 2
NS = 16
NW = NC * NS

B_PER_W = BATCH // NW
CHUNK = 128
NB = B_PER_W // CHUNK
NCHUNK = WINDOW * NB


def _gather_body(idx_hbm, table_hbm, out_hbm, idx_v, rows_v, sem):
  wid = lax.axis_index("s") * NC + lax.axis_index("c")
  pltpu.sync_copy(idx_hbm.at[wid], idx_v)
  for w in range(WINDOW):
    for c in range(NB):
      pltpu.async_copy(
          table_hbm.at[idx_v.at[w * NB + c]], rows_v, sem).wait()
      pltpu.sync_copy(
          rows_v, out_hbm.at[w, pl.ds(wid * B_PER_W + c * CHUNK, CHUNK)])


_sc_gather = functools.partial(
    pl.kernel,
    out_type=jax.ShapeDtypeStruct((WINDOW, BATCH, EMB), jnp.float32),
    mesh=plsc.VectorSubcoreMesh(core_axis_name="c", subcore_axis_name="s"),
    scratch_types=[
        pltpu.VMEM((NCHUNK, CHUNK), jnp.int32),
        pltpu.VMEM((CHUNK, EMB), jnp.float32),
        pltpu.SemaphoreType.DMA,
    ],
)(_gather_body)


BM = 1024


def _matmul_body(g_ref, w_ref, b_ref, out_ref):
  acc = b_ref[...]
  for w in range(WINDOW):
    acc = acc + lax.dot_general(
        g_ref[w], w_ref[w],
        dimension_numbers=(((1,), (1,)), ((), ())),
        preferred_element_type=jnp.float32,
    )
  out_ref[...] = acc


def _tc_matmul(g, wr, b2d):
  return pl.pallas_call(
      _matmul_body,
      grid=(BATCH // BM,),
      in_specs=[
          pl.BlockSpec((WINDOW, BM, EMB), lambda i: (0, i, 0)),
          pl.BlockSpec((WINDOW, N_CLASS, EMB), lambda i: (0, 0, 0)),
          pl.BlockSpec((1, N_CLASS), lambda i: (0, 0)),
      ],
      out_specs=pl.BlockSpec((BM, N_CLASS), lambda i: (i, 0)),
      out_shape=jax.ShapeDtypeStruct((BATCH, N_CLASS), jnp.float32),
  )(g, wr, b2d)


@jax.jit
def kernel(x, table, W, b):
  idx = (x.astype(jnp.int32).T
         .reshape(WINDOW, NW, NB, CHUNK)
         .transpose(1, 0, 2, 3)
         .reshape(NW, NCHUNK, CHUNK))
  g = _sc_gather(idx, table)
  wr = W.reshape(N_CLASS, WINDOW, EMB).transpose(1, 0, 2)
  return _tc_matmul(g, wr, b.reshape(1, N_CLASS))


# speedup vs baseline: 5.5653x; 1.1736x over previous
import functools

import jax
import jax.numpy as jnp
from jax import lax
from jax.experimental import pallas as pl
from jax.experimental.pallas import tpu as pltpu
from jax.experimental.pallas import tpu_sc as plsc

VOCAB_P1 = 100001
EMB = 128
BATCH = 16384
WINDOW = 5
N_CLASS = 50

NC = 2
NS = 16
NW = NC * NS

B_PER_W = BATCH // NW
CHUNK = 128
NB = B_PER_W // CHUNK
NCHUNK = WINDOW * NB


K = 2
NGRP = NCHUNK // K


def _gather_body(idx_hbm, table_hbm, out_hbm, idx_v,
                 a0, a1, b0, b1, gsa, gsb, wsa, wsb):
  wid = lax.axis_index("s") * NC + lax.axis_index("c")
  base = wid * B_PER_W
  pltpu.sync_copy(idx_hbm.at[wid], idx_v)
  bufs = [(a0, a1, gsa, wsa), (b0, b1, gsb, wsb)]

  def fire_gathers(grp):
    r0, r1, gs, _ = bufs[grp % 2]
    j = grp * K
    return [pltpu.async_copy(table_hbm.at[idx_v.at[j]], r0, gs),
            pltpu.async_copy(table_hbm.at[idx_v.at[j + 1]], r1, gs)]

  def fire_writes(grp):
    _, _, _, ws = bufs[grp % 2]
    hs = []
    for k, r in ((0, bufs[grp % 2][0]), (1, bufs[grp % 2][1])):
      j = grp * K + k
      w, c = j // NB, j % NB
      hs.append(pltpu.async_copy(
          r, out_hbm.at[w, pl.ds(base + c * CHUNK, CHUNK)], ws))
    return hs

  g_handles = {0: fire_gathers(0)}
  w_handles = {}
  for grp in range(NGRP):
    if grp + 1 < NGRP:
      if grp >= 1:
        for h in w_handles[grp - 1]:
          h.wait()
      g_handles[grp + 1] = fire_gathers(grp + 1)
    for h in g_handles[grp]:
      h.wait()
    w_handles[grp] = fire_writes(grp)
  for grp in (NGRP - 2, NGRP - 1):
    for h in w_handles[grp]:
      h.wait()


_sc_gather = functools.partial(
    pl.kernel,
    out_type=jax.ShapeDtypeStruct((WINDOW, BATCH, EMB), jnp.float32),
    mesh=plsc.VectorSubcoreMesh(core_axis_name="c", subcore_axis_name="s"),
    scratch_types=[
        pltpu.VMEM((NCHUNK, CHUNK), jnp.int32),
        pltpu.VMEM((CHUNK, EMB), jnp.float32),
        pltpu.VMEM((CHUNK, EMB), jnp.float32),
        pltpu.VMEM((CHUNK, EMB), jnp.float32),
        pltpu.VMEM((CHUNK, EMB), jnp.float32),
        pltpu.SemaphoreType.DMA,
        pltpu.SemaphoreType.DMA,
        pltpu.SemaphoreType.DMA,
        pltpu.SemaphoreType.DMA,
    ],
)(_gather_body)


BM = 1024


def _matmul_body(g_ref, w_ref, b_ref, out_ref):
  acc = b_ref[...]
  for w in range(WINDOW):
    acc = acc + lax.dot_general(
        g_ref[w], w_ref[w],
        dimension_numbers=(((1,), (1,)), ((), ())),
        preferred_element_type=jnp.float32,
    )
  out_ref[...] = acc


def _tc_matmul(g, wr, b2d):
  return pl.pallas_call(
      _matmul_body,
      grid=(BATCH // BM,),
      in_specs=[
          pl.BlockSpec((WINDOW, BM, EMB), lambda i: (0, i, 0)),
          pl.BlockSpec((WINDOW, N_CLASS, EMB), lambda i: (0, 0, 0)),
          pl.BlockSpec((1, N_CLASS), lambda i: (0, 0)),
      ],
      out_specs=pl.BlockSpec((BM, N_CLASS), lambda i: (i, 0)),
      out_shape=jax.ShapeDtypeStruct((BATCH, N_CLASS), jnp.float32),
  )(g, wr, b2d)


@jax.jit
def kernel(x, table, W, b):
  idx = (x.astype(jnp.int32).T
         .reshape(WINDOW, NW, NB, CHUNK)
         .transpose(1, 0, 2, 3)
         .reshape(NW, NCHUNK, CHUNK))
  g = _sc_gather(idx, table)
  wr = W.reshape(N_CLASS, WINDOW, EMB).transpose(1, 0, 2)
  return _tc_matmul(g, wr, b.reshape(1, N_CLASS))


# speedup vs baseline: 5.8352x; 1.0485x over previous
import functools

import jax
import jax.numpy as jnp
from jax import lax
from jax.experimental import pallas as pl
from jax.experimental.pallas import tpu as pltpu
from jax.experimental.pallas import tpu_sc as plsc

VOCAB_P1 = 100001
EMB = 128
BATCH = 16384
WINDOW = 5
N_CLASS = 50

NC = 2
NS = 16
NW = NC * NS

B_PER_W = BATCH // NW
CHUNK = 128
NB = B_PER_W // CHUNK
NCHUNK = WINDOW * NB


K = 2
NGRP = NCHUNK // K


def _gather_body(idx_hbm, table_hbm, out_hbm, idx_v,
                 a0, a1, b0, b1, gsa, gsb, wsa, wsb):
  wid = lax.axis_index("s") * NC + lax.axis_index("c")
  base = wid * B_PER_W
  pltpu.sync_copy(idx_hbm.at[wid], idx_v)
  bufs = [(a0, a1, gsa, wsa), (b0, b1, gsb, wsb)]

  def fire_gathers(grp):
    r0, r1, gs, _ = bufs[grp % 2]
    j = grp * K
    return [pltpu.async_copy(table_hbm.at[idx_v.at[j]], r0, gs),
            pltpu.async_copy(table_hbm.at[idx_v.at[j + 1]], r1, gs)]

  def fire_writes(grp):
    _, _, _, ws = bufs[grp % 2]
    hs = []
    for k, r in ((0, bufs[grp % 2][0]), (1, bufs[grp % 2][1])):
      j = grp * K + k
      w, c = j // NB, j % NB
      hs.append(pltpu.async_copy(
          r, out_hbm.at[w, pl.ds(base + c * CHUNK, CHUNK)], ws))
    return hs

  g_handles = {0: fire_gathers(0)}
  w_handles = {}
  for grp in range(NGRP):
    if grp + 1 < NGRP:
      if grp >= 1:
        for h in w_handles[grp - 1]:
          h.wait()
      g_handles[grp + 1] = fire_gathers(grp + 1)
    for h in g_handles[grp]:
      h.wait()
    w_handles[grp] = fire_writes(grp)
  for grp in (NGRP - 2, NGRP - 1):
    for h in w_handles[grp]:
      h.wait()


_sc_gather = functools.partial(
    pl.kernel,
    out_type=jax.ShapeDtypeStruct((WINDOW, BATCH, EMB), jnp.float32),
    mesh=plsc.VectorSubcoreMesh(core_axis_name="c", subcore_axis_name="s"),
    scratch_types=[
        pltpu.VMEM((NCHUNK, CHUNK), jnp.int32),
        pltpu.VMEM((CHUNK, EMB), jnp.float32),
        pltpu.VMEM((CHUNK, EMB), jnp.float32),
        pltpu.VMEM((CHUNK, EMB), jnp.float32),
        pltpu.VMEM((CHUNK, EMB), jnp.float32),
        pltpu.SemaphoreType.DMA,
        pltpu.SemaphoreType.DMA,
        pltpu.SemaphoreType.DMA,
        pltpu.SemaphoreType.DMA,
    ],
)(_gather_body)


BM = 2048


def _matmul_body(g_ref, w_ref, b_ref, out_ref):
  acc = b_ref[...]
  for w in range(WINDOW):
    acc = acc + lax.dot_general(
        g_ref[w], w_ref[w],
        dimension_numbers=(((1,), (1,)), ((), ())),
        preferred_element_type=jnp.float32,
    )
  out_ref[...] = acc


def _tc_matmul(g, wr, b2d):
  return pl.pallas_call(
      _matmul_body,
      grid=(BATCH // BM,),
      in_specs=[
          pl.BlockSpec((WINDOW, BM, EMB), lambda i: (0, i, 0)),
          pl.BlockSpec((WINDOW, N_CLASS, EMB), lambda i: (0, 0, 0)),
          pl.BlockSpec((1, N_CLASS), lambda i: (0, 0)),
      ],
      out_specs=pl.BlockSpec((BM, N_CLASS), lambda i: (i, 0)),
      out_shape=jax.ShapeDtypeStruct((BATCH, N_CLASS), jnp.float32),
  )(g, wr, b2d)


@jax.jit
def kernel(x, table, W, b):
  idx = (x.astype(jnp.int32).T
         .reshape(WINDOW, NW, NB, CHUNK)
         .transpose(1, 0, 2, 3)
         .reshape(NW, NCHUNK, CHUNK))
  g = _sc_gather(idx, table)
  wr = W.reshape(N_CLASS, WINDOW, EMB).transpose(1, 0, 2)
  return _tc_matmul(g, wr, b.reshape(1, N_CLASS))
